# baseline (speedup 1.0000x reference)
import functools

import jax
import jax.numpy as jnp
from jax import lax
from jax.experimental import pallas as pl
from jax.experimental.pallas import tpu as pltpu
from jax.experimental.pallas import tpu_sc as plsc

NC, NS, L = 2, 16, 16
NW = NC * NS
V = 21
VP = 32


def _sqrt16(x):
    i = plsc.bitcast(x, jnp.int32)
    y = plsc.bitcast(jnp.int32(0x5F3759DF) - (i >> 1), jnp.float32)
    hx = 0.5 * x
    y = y * (1.5 - hx * y * y)
    y = y * (1.5 - hx * y * y)
    return x * y


@functools.cache
def _make_score(B):
    bpw = B // NW
    ni = bpw // L
    mesh = plsc.VectorSubcoreMesh(core_axis_name="c", subcore_axis_name="s")

    @functools.partial(
        pl.kernel,
        out_type=(jax.ShapeDtypeStruct((12, B), jnp.float32),
                  jax.ShapeDtypeStruct((B,), jnp.float32)),
        mesh=mesh,
        compiler_params=pltpu.CompilerParams(
            needs_layout_passes=False,
            skip_device_barrier=True,
            disable_bounds_checks=True,
        ),
        scratch_types=[
            pltpu.VMEM((3, V), jnp.float32),
            pltpu.VMEM((5, bpw), jnp.int32),
            pltpu.VMEM((2, bpw), jnp.int32),
            pltpu.VMEM((V, VP), jnp.float32),
            pltpu.VMEM((12, bpw), jnp.float32),
            pltpu.VMEM((bpw,), jnp.float32),
            pltpu.SemaphoreType.DMA,
            pltpu.SemaphoreType.DMA,
            pltpu.SemaphoreType.DMA,
        ],
    )
    def _score(e_hbm, g_hbm, r2_hbm, rank_hbm, rate_hbm,
               e_v, g_v, r2_v, s_v, outr_v, outt_v,
               sem_e, sem_g, sem_r):
        wid = lax.axis_index("s") * NC + lax.axis_index("c")
        base = wid * bpw
        ce = pltpu.async_copy(e_hbm, e_v, sem_e)
        cg = pltpu.async_copy(g_hbm.at[:, pl.ds(base, bpw)], g_v, sem_g)
        cr = pltpu.async_copy(r2_hbm.at[:, pl.ds(base, bpw)], r2_v, sem_r)
        iota = lax.iota(jnp.int32, L)
        ce.wait()

        @plsc.parallel_loop(0, V, 1, unroll=3)
        def table_row(q):
            qsplat = jnp.full((L,), q, jnp.int32)
            qk = [plsc.load_gather(e_v, [jnp.full((L,), k, jnp.int32), qsplat])
                  for k in range(3)]
            for h in range(2):
                r = h * L + iota
                rc = jnp.minimum(r, V - 1) if h else r
                d2 = jnp.full((L,), 1e-12, jnp.float32)
                for k in range(3):
                    diff = qk[k] - plsc.load_gather(
                        e_v, [jnp.full((L,), k, jnp.int32), rc])
                    d2 = d2 + diff * diff
                s_v[q, pl.ds(h * L, L)] = (
                    jnp.exp(-10.0 * _sqrt16(d2)) + 0.001)

        cg.wait()
        cr.wait()

        @plsc.parallel_loop(0, ni, 1, unroll=4)
        def body(i):
            off = i * L
            q = g_v[0, pl.ds(off, L)]
            s = [plsc.load_gather(s_v, [q, g_v[j, pl.ds(off, L)]])
                 for j in range(1, 5)]
            sr = plsc.load_gather(
                s_v, [r2_v[0, pl.ds(off, L)], r2_v[1, pl.ds(off, L)]])
            total = ((s[0] + s[1]) + s[2]) + s[3]
            a = [total, total - s[0], total - s[1], total - s[2],
                 total - s[3], 1.0 + jnp.exp(-sr)]
            pre = [a[0]]
            for k in range(1, 5):
                pre.append(pre[-1] * a[k])
            suf = [a[5]]
            for k in range(4, 0, -1):
                suf.append(suf[-1] * a[k])
            inv_p = 1.0 / (pre[4] * a[5])
            it = suf[4] * inv_p
            dn = [pre[k - 1] * suf[4 - k] * inv_p for k in range(1, 5)]
            u = [sj * it for sj in s]
            slot = 0
            for x in range(4):
                for y in range(4):
                    if x == y:
                        continue
                    outr_v[slot, pl.ds(off, L)] = u[x] * (s[y] * dn[x])
                    slot += 1
            outt_v[pl.ds(off, L)] = pre[4] * inv_p

        pltpu.sync_copy(outr_v, rank_hbm.at[:, pl.ds(base, bpw)])
        pltpu.sync_copy(outt_v, rate_hbm.at[pl.ds(base, bpw)])

    return _score


def kernel(given4rank2_stimulus_set, rate2_stimulus_set, percept_embeddings):
    B = given4rank2_stimulus_set.shape[0]
    rank12, rate_flat = _make_score(B)(
        percept_embeddings.T,
        given4rank2_stimulus_set.T,
        rate2_stimulus_set.T)
    return rank12.T, rate_flat.reshape(B, 1)
